# baseline (speedup 1.0000x reference)
import functools

import jax
import jax.numpy as jnp
from jax.experimental import pallas as pl
from jax.experimental.pallas import tpu as pltpu

NEG_SLOPE = 0.01
BN_EPS = 1e-5
EDGE_UNROLL = 128
N_CHAINS = 5


def _lrelu(z):
    return jnp.where(z >= 0.0, z, NEG_SLOPE * z)


def _transform_kernel(x_ref, w_ref, d_ref, o_ref):
    g = jnp.dot(x_ref[...], w_ref[...], preferred_element_type=jnp.float32)
    o_ref[...] = d_ref[...] * g


def _agg_kernel(pk_ref, u_ref, o_ref, *accs, n_iters, half, shift, mask):
    c = pl.program_id(0)
    for a in accs:
        a[...] = jnp.zeros_like(a)
    base0 = c * half

    def body(j, carry):
        base = base0 + j * EDGE_UNROLL
        for i in range(EDGE_UNROLL):
            a = accs[i % N_CHAINS]
            p = pk_ref[base + i]
            d = p >> shift
            s = p & mask
            a[d, 0] = a[d, 0] + u_ref[s, 0]
        return carry

    jax.lax.fori_loop(0, n_iters, body, 0)
    parts = [a[:, 0, :] for a in accs]
    while len(parts) > 1:
        parts = ([parts[i] + parts[i + 1] for i in range(0, len(parts) - 1, 2)]
                 + ([parts[-1]] if len(parts) % 2 else []))
    o_ref[0] = parts[0].astype(o_ref.dtype)


def _finalize0_kernel(p0_ref, p1_ref, u_ref, d_ref, b_ref, o_ref):
    m = (p0_ref[0].astype(jnp.float32) + p1_ref[0].astype(jnp.float32)
         + u_ref[...])
    z = d_ref[...] * m + b_ref[...]
    o_ref[...] = d_ref[...] * _lrelu(z)


def _finalize_kernel(p0_ref, p1_ref, u_ref, d_ref, w_ref, b_ref, o_ref, *,
                     rescale):
    m = (p0_ref[0].astype(jnp.float32) + p1_ref[0].astype(jnp.float32)
         + u_ref[...])
    z = d_ref[...] * jnp.dot(m, w_ref[...],
                             preferred_element_type=jnp.float32) + b_ref[...]
    h = _lrelu(z)
    if rescale:
        h = d_ref[...] * h
    o_ref[...] = h.astype(o_ref.dtype)


def _readout_kernel(p_ref, h_ref, w_ref, b_ref, o_ref):
    pooled = jnp.dot(p_ref[...], h_ref[...], preferred_element_type=jnp.float32)
    o_ref[...] = (jnp.dot(pooled, w_ref[...], preferred_element_type=jnp.float32)
                  + b_ref[...]).astype(o_ref.dtype)


def _transform(x, w, d_col):
    n, cin = x.shape
    cout = w.shape[1]
    tm = 2048
    return pl.pallas_call(
        _transform_kernel,
        out_shape=jax.ShapeDtypeStruct((n, cout), jnp.float32),
        grid=(n // tm,),
        in_specs=[
            pl.BlockSpec((tm, cin), lambda i: (i, 0)),
            pl.BlockSpec((cin, cout), lambda i: (0, 0)),
            pl.BlockSpec((tm, 1), lambda i: (i, 0)),
        ],
        out_specs=pl.BlockSpec((tm, cout), lambda i: (i, 0)),
        compiler_params=pltpu.CompilerParams(
            dimension_semantics=("parallel",)),
    )(x, w, d_col)


def _aggregate(pk, u):
    n, c = u.shape
    n_e = pk.shape[0]
    half = n_e // 2
    u3 = u.reshape(n, 1, c)
    return pl.pallas_call(
        functools.partial(_agg_kernel, n_iters=half // EDGE_UNROLL, half=half,
                          shift=(n - 1).bit_length(), mask=n - 1),
        out_shape=jax.ShapeDtypeStruct((2, n, c), jnp.bfloat16),
        grid_spec=pltpu.PrefetchScalarGridSpec(
            num_scalar_prefetch=1,
            grid=(2,),
            in_specs=[pl.BlockSpec((n, 1, c), lambda i, pk: (0, 0, 0))],
            out_specs=pl.BlockSpec((1, n, c), lambda i, pk: (i, 0, 0)),
            scratch_shapes=[pltpu.VMEM((n, 1, c), jnp.float32)] * N_CHAINS,
        ),
        compiler_params=pltpu.CompilerParams(
            dimension_semantics=("parallel",),
            vmem_limit_bytes=58 * 1024 * 1024,
        ),
    )(pk, u3)


def _finalize0(partials, u, d_col, b):
    n, c = u.shape
    tm = 512
    return pl.pallas_call(
        _finalize0_kernel,
        out_shape=jax.ShapeDtypeStruct((n, c), jnp.float32),
        grid=(n // tm,),
        in_specs=[
            pl.BlockSpec((1, tm, c), lambda i: (0, i, 0)),
            pl.BlockSpec((1, tm, c), lambda i: (1, i, 0)),
            pl.BlockSpec((tm, c), lambda i: (i, 0)),
            pl.BlockSpec((tm, 1), lambda i: (i, 0)),
            pl.BlockSpec((1, c), lambda i: (0, 0)),
        ],
        out_specs=pl.BlockSpec((tm, c), lambda i: (i, 0)),
        compiler_params=pltpu.CompilerParams(
            dimension_semantics=("parallel",)),
    )(partials, partials, u, d_col, b)


def _finalize(partials, u, d_col, w, b, *, rescale, out_dtype):
    n, c = u.shape
    tm = 512
    return pl.pallas_call(
        functools.partial(_finalize_kernel, rescale=rescale),
        out_shape=jax.ShapeDtypeStruct((n, c), out_dtype),
        grid=(n // tm,),
        in_specs=[
            pl.BlockSpec((1, tm, c), lambda i: (0, i, 0)),
            pl.BlockSpec((1, tm, c), lambda i: (1, i, 0)),
            pl.BlockSpec((tm, c), lambda i: (i, 0)),
            pl.BlockSpec((tm, 1), lambda i: (i, 0)),
            pl.BlockSpec((c, c), lambda i: (0, 0)),
            pl.BlockSpec((1, c), lambda i: (0, 0)),
        ],
        out_specs=pl.BlockSpec((tm, c), lambda i: (i, 0)),
        compiler_params=pltpu.CompilerParams(
            dimension_semantics=("parallel",)),
    )(partials, partials, u, d_col, w, b)


def _readout(pool, h, w_lin, b_lin):
    g = pool.shape[0]
    o = w_lin.shape[1]
    vspec = pl.BlockSpec(memory_space=pltpu.MemorySpace.VMEM)
    return pl.pallas_call(
        _readout_kernel,
        out_shape=jax.ShapeDtypeStruct((g, o), jnp.float32),
        in_specs=[vspec] * 4,
        out_specs=vspec,
        compiler_params=pltpu.CompilerParams(
            vmem_limit_bytes=32 * 1024 * 1024),
    )(pool, h, w_lin, b_lin)


def kernel(x, edge_index, batch,
           w0, b0, gamma0, beta0, run_mean0, run_var0,
           w1, b1, gamma1, beta1, run_mean1, run_var1,
           w2, b2, gamma2, beta2, run_mean2, run_var2,
           lin_w, lin_b):
    n_nodes = x.shape[0]
    num_graphs = 32
    out_ch = lin_w.shape[1]

    src, dst = edge_index[0], edge_index[1]
    deg = jnp.zeros((n_nodes,), jnp.float32).at[dst].add(1.0) + 1.0
    d_inv_sqrt = 1.0 / jnp.sqrt(deg)
    d_col = d_inv_sqrt.reshape(n_nodes, 1)

    pk = (dst << (n_nodes - 1).bit_length()) | src

    ws, bs = [], []
    for (w, b, gamma, beta, mean, var) in (
            (w0, b0, gamma0, beta0, run_mean0, run_var0),
            (w1, b1, gamma1, beta1, run_mean1, run_var1),
            (w2, b2, gamma2, beta2, run_mean2, run_var2)):
        scale = gamma / jnp.sqrt(var + BN_EPS)
        ws.append((w * scale[None, :]).astype(jnp.float32))
        bs.append(((b - mean) * scale + beta).reshape(1, -1).astype(jnp.float32))

    pool = (batch[None, :] == jnp.arange(num_graphs, dtype=batch.dtype)[:, None]
            ).astype(jnp.bfloat16)

    u = _transform(x, ws[0], d_col)
    partials = _aggregate(pk, u)
    u = _finalize0(partials, u, d_col, bs[0])
    partials = _aggregate(pk, u)
    u = _finalize(partials, u, d_col, ws[1], bs[1],
                  rescale=True, out_dtype=jnp.float32)
    partials = _aggregate(pk, u)
    h3 = _finalize(partials, u, d_col, ws[2], bs[2],
                   rescale=False, out_dtype=jnp.bfloat16)
    out = _readout(pool, h3, lin_w.astype(jnp.float32),
                   lin_b.reshape(1, -1).astype(jnp.float32))
    return out[:, :out_ch]


# speedup vs baseline: 1.1466x; 1.1466x over previous
import functools

import jax
import jax.numpy as jnp
from jax.experimental import pallas as pl
from jax.experimental.pallas import tpu as pltpu

NEG_SLOPE = 0.01
BN_EPS = 1e-5
EDGE_UNROLL = 128
N_CHAINS = 4


def _lrelu(z):
    return jnp.where(z >= 0.0, z, NEG_SLOPE * z)


def _transform_kernel(x_ref, w_ref, d_ref, o_ref):
    g = jnp.dot(x_ref[...], w_ref[...], preferred_element_type=jnp.float32)
    o_ref[...] = d_ref[...] * g


def _agg_kernel(pk_ref, u_ref, o_ref, *accs, n_iters, half, shift, mask):
    c = pl.program_id(0)
    for a in accs:
        a[...] = jnp.zeros_like(a)
    base0 = c * half

    def body(j, carry):
        base = base0 + j * EDGE_UNROLL
        for i in range(EDGE_UNROLL):
            a = accs[i % N_CHAINS]
            p = pk_ref[base + i]
            d = p >> shift
            s = p & mask
            a[d, 0] = a[d, 0] + u_ref[s, 0]
        return carry

    jax.lax.fori_loop(0, n_iters, body, 0)
    parts = [a[:, 0, :] for a in accs]
    while len(parts) > 1:
        parts = ([parts[i] + parts[i + 1] for i in range(0, len(parts) - 1, 2)]
                 + ([parts[-1]] if len(parts) % 2 else []))
    o_ref[0] = parts[0].astype(o_ref.dtype)


def _finalize0_kernel(p0_ref, p1_ref, u_ref, d_ref, b_ref, o_ref):
    m = (p0_ref[0].astype(jnp.float32) + p1_ref[0].astype(jnp.float32)
         + u_ref[...])
    z = d_ref[...] * m + b_ref[...]
    o_ref[...] = d_ref[...] * _lrelu(z)


def _finalize_kernel(p0_ref, p1_ref, u_ref, d_ref, w_ref, b_ref, o_ref, *,
                     rescale):
    m = (p0_ref[0].astype(jnp.float32) + p1_ref[0].astype(jnp.float32)
         + u_ref[...])
    z = d_ref[...] * jnp.dot(m, w_ref[...],
                             preferred_element_type=jnp.float32) + b_ref[...]
    h = _lrelu(z)
    if rescale:
        h = d_ref[...] * h
    o_ref[...] = h.astype(o_ref.dtype)


def _readout_kernel(p_ref, h_ref, w_ref, b_ref, o_ref):
    pooled = jnp.dot(p_ref[...], h_ref[...], preferred_element_type=jnp.float32)
    o_ref[...] = (jnp.dot(pooled, w_ref[...], preferred_element_type=jnp.float32)
                  + b_ref[...]).astype(o_ref.dtype)


def _transform(x, w, d_col):
    n, cin = x.shape
    cout = w.shape[1]
    tm = 2048
    return pl.pallas_call(
        _transform_kernel,
        out_shape=jax.ShapeDtypeStruct((n, cout), jnp.float32),
        grid=(n // tm,),
        in_specs=[
            pl.BlockSpec((tm, cin), lambda i: (i, 0)),
            pl.BlockSpec((cin, cout), lambda i: (0, 0)),
            pl.BlockSpec((tm, 1), lambda i: (i, 0)),
        ],
        out_specs=pl.BlockSpec((tm, cout), lambda i: (i, 0)),
        compiler_params=pltpu.CompilerParams(
            dimension_semantics=("parallel",)),
    )(x, w, d_col)


def _aggregate(pk, u):
    n, c = u.shape
    n_e = pk.shape[0]
    half = n_e // 2
    u3 = u.reshape(n, 1, c)
    return pl.pallas_call(
        functools.partial(_agg_kernel, n_iters=half // EDGE_UNROLL, half=half,
                          shift=(n - 1).bit_length(), mask=n - 1),
        out_shape=jax.ShapeDtypeStruct((2, n, c), jnp.float32),
        grid_spec=pltpu.PrefetchScalarGridSpec(
            num_scalar_prefetch=1,
            grid=(2,),
            in_specs=[pl.BlockSpec((n, 1, c), lambda i, pk: (0, 0, 0))],
            out_specs=pl.BlockSpec((1, n, c), lambda i, pk: (i, 0, 0)),
            scratch_shapes=[pltpu.VMEM((n, 1, c), jnp.float32)] * N_CHAINS,
        ),
        compiler_params=pltpu.CompilerParams(
            dimension_semantics=("parallel",),
            vmem_limit_bytes=58 * 1024 * 1024,
        ),
    )(pk, u3)


def _finalize0(partials, u, d_col, b):
    n, c = u.shape
    tm = 512
    return pl.pallas_call(
        _finalize0_kernel,
        out_shape=jax.ShapeDtypeStruct((n, c), jnp.float32),
        grid=(n // tm,),
        in_specs=[
            pl.BlockSpec((1, tm, c), lambda i: (0, i, 0)),
            pl.BlockSpec((1, tm, c), lambda i: (1, i, 0)),
            pl.BlockSpec((tm, c), lambda i: (i, 0)),
            pl.BlockSpec((tm, 1), lambda i: (i, 0)),
            pl.BlockSpec((1, c), lambda i: (0, 0)),
        ],
        out_specs=pl.BlockSpec((tm, c), lambda i: (i, 0)),
        compiler_params=pltpu.CompilerParams(
            dimension_semantics=("parallel",)),
    )(partials, partials, u, d_col, b)


def _finalize(partials, u, d_col, w, b, *, rescale, out_dtype):
    n, c = u.shape
    tm = 512
    return pl.pallas_call(
        functools.partial(_finalize_kernel, rescale=rescale),
        out_shape=jax.ShapeDtypeStruct((n, c), out_dtype),
        grid=(n // tm,),
        in_specs=[
            pl.BlockSpec((1, tm, c), lambda i: (0, i, 0)),
            pl.BlockSpec((1, tm, c), lambda i: (1, i, 0)),
            pl.BlockSpec((tm, c), lambda i: (i, 0)),
            pl.BlockSpec((tm, 1), lambda i: (i, 0)),
            pl.BlockSpec((c, c), lambda i: (0, 0)),
            pl.BlockSpec((1, c), lambda i: (0, 0)),
        ],
        out_specs=pl.BlockSpec((tm, c), lambda i: (i, 0)),
        compiler_params=pltpu.CompilerParams(
            dimension_semantics=("parallel",)),
    )(partials, partials, u, d_col, w, b)


def _readout(pool, h, w_lin, b_lin):
    g = pool.shape[0]
    o = w_lin.shape[1]
    vspec = pl.BlockSpec(memory_space=pltpu.MemorySpace.VMEM)
    return pl.pallas_call(
        _readout_kernel,
        out_shape=jax.ShapeDtypeStruct((g, o), jnp.float32),
        in_specs=[vspec] * 4,
        out_specs=vspec,
        compiler_params=pltpu.CompilerParams(
            vmem_limit_bytes=32 * 1024 * 1024),
    )(pool, h, w_lin, b_lin)


def kernel(x, edge_index, batch,
           w0, b0, gamma0, beta0, run_mean0, run_var0,
           w1, b1, gamma1, beta1, run_mean1, run_var1,
           w2, b2, gamma2, beta2, run_mean2, run_var2,
           lin_w, lin_b):
    n_nodes = x.shape[0]
    num_graphs = 32
    out_ch = lin_w.shape[1]

    src, dst = edge_index[0], edge_index[1]
    deg = jnp.zeros((n_nodes,), jnp.float32).at[dst].add(1.0) + 1.0
    d_inv_sqrt = 1.0 / jnp.sqrt(deg)
    d_col = d_inv_sqrt.reshape(n_nodes, 1)

    pk = (dst << (n_nodes - 1).bit_length()) | src

    ws, bs = [], []
    for (w, b, gamma, beta, mean, var) in (
            (w0, b0, gamma0, beta0, run_mean0, run_var0),
            (w1, b1, gamma1, beta1, run_mean1, run_var1),
            (w2, b2, gamma2, beta2, run_mean2, run_var2)):
        scale = gamma / jnp.sqrt(var + BN_EPS)
        ws.append((w * scale[None, :]).astype(jnp.float32))
        bs.append(((b - mean) * scale + beta).reshape(1, -1).astype(jnp.float32))

    pool = (batch[None, :] == jnp.arange(num_graphs, dtype=batch.dtype)[:, None]
            ).astype(jnp.bfloat16)

    u = _transform(x, ws[0], d_col)
    partials = _aggregate(pk, u)
    u = _finalize0(partials, u, d_col, bs[0])
    partials = _aggregate(pk, u)
    u = _finalize(partials, u, d_col, ws[1], bs[1],
                  rescale=True, out_dtype=jnp.float32)
    partials = _aggregate(pk, u)
    h3 = _finalize(partials, u, d_col, ws[2], bs[2],
                   rescale=False, out_dtype=jnp.bfloat16)
    out = _readout(pool, h3, lin_w.astype(jnp.float32),
                   lin_b.reshape(1, -1).astype(jnp.float32))
    return out[:, :out_ch]


# speedup vs baseline: 1.1507x; 1.0036x over previous
import functools

import jax
import jax.numpy as jnp
from jax.experimental import pallas as pl
from jax.experimental.pallas import tpu as pltpu

NEG_SLOPE = 0.01
BN_EPS = 1e-5
EDGE_UNROLL = 256
N_CHAINS = 4


def _lrelu(z):
    return jnp.where(z >= 0.0, z, NEG_SLOPE * z)


def _transform_kernel(x_ref, w_ref, d_ref, o_ref):
    g = jnp.dot(x_ref[...], w_ref[...], preferred_element_type=jnp.float32)
    o_ref[...] = d_ref[...] * g


def _agg_kernel(pk_ref, u_ref, o_ref, *accs, n_iters, half, shift, mask):
    c = pl.program_id(0)
    for a in accs:
        a[...] = jnp.zeros_like(a)
    base0 = c * half

    def body(j, carry):
        base = base0 + j * EDGE_UNROLL
        for i in range(EDGE_UNROLL):
            a = accs[i % N_CHAINS]
            p = pk_ref[base + i]
            d = p >> shift
            s = p & mask
            a[d, 0] = a[d, 0] + u_ref[s, 0]
        return carry

    jax.lax.fori_loop(0, n_iters, body, 0)
    parts = [a[:, 0, :] for a in accs]
    while len(parts) > 1:
        parts = ([parts[i] + parts[i + 1] for i in range(0, len(parts) - 1, 2)]
                 + ([parts[-1]] if len(parts) % 2 else []))
    o_ref[0] = parts[0].astype(o_ref.dtype)


def _finalize0_kernel(p0_ref, p1_ref, u_ref, d_ref, b_ref, o_ref):
    m = (p0_ref[0].astype(jnp.float32) + p1_ref[0].astype(jnp.float32)
         + u_ref[...])
    z = d_ref[...] * m + b_ref[...]
    o_ref[...] = d_ref[...] * _lrelu(z)


def _finalize_kernel(p0_ref, p1_ref, u_ref, d_ref, w_ref, b_ref, o_ref, *,
                     rescale):
    m = (p0_ref[0].astype(jnp.float32) + p1_ref[0].astype(jnp.float32)
         + u_ref[...])
    z = d_ref[...] * jnp.dot(m, w_ref[...],
                             preferred_element_type=jnp.float32) + b_ref[...]
    h = _lrelu(z)
    if rescale:
        h = d_ref[...] * h
    o_ref[...] = h.astype(o_ref.dtype)


def _readout_kernel(p_ref, h_ref, w_ref, b_ref, o_ref):
    pooled = jnp.dot(p_ref[...], h_ref[...], preferred_element_type=jnp.float32)
    o_ref[...] = (jnp.dot(pooled, w_ref[...], preferred_element_type=jnp.float32)
                  + b_ref[...]).astype(o_ref.dtype)


def _transform(x, w, d_col):
    n, cin = x.shape
    cout = w.shape[1]
    tm = 2048
    return pl.pallas_call(
        _transform_kernel,
        out_shape=jax.ShapeDtypeStruct((n, cout), jnp.float32),
        grid=(n // tm,),
        in_specs=[
            pl.BlockSpec((tm, cin), lambda i: (i, 0)),
            pl.BlockSpec((cin, cout), lambda i: (0, 0)),
            pl.BlockSpec((tm, 1), lambda i: (i, 0)),
        ],
        out_specs=pl.BlockSpec((tm, cout), lambda i: (i, 0)),
        compiler_params=pltpu.CompilerParams(
            dimension_semantics=("parallel",)),
    )(x, w, d_col)


def _aggregate(pk, u):
    n, c = u.shape
    n_e = pk.shape[0]
    half = n_e // 2
    u3 = u.reshape(n, 1, c)
    return pl.pallas_call(
        functools.partial(_agg_kernel, n_iters=half // EDGE_UNROLL, half=half,
                          shift=(n - 1).bit_length(), mask=n - 1),
        out_shape=jax.ShapeDtypeStruct((2, n, c), jnp.float32),
        grid_spec=pltpu.PrefetchScalarGridSpec(
            num_scalar_prefetch=1,
            grid=(2,),
            in_specs=[pl.BlockSpec((n, 1, c), lambda i, pk: (0, 0, 0))],
            out_specs=pl.BlockSpec((1, n, c), lambda i, pk: (i, 0, 0)),
            scratch_shapes=[pltpu.VMEM((n, 1, c), jnp.float32)] * N_CHAINS,
        ),
        compiler_params=pltpu.CompilerParams(
            dimension_semantics=("parallel",),
            vmem_limit_bytes=58 * 1024 * 1024,
        ),
    )(pk, u3)


def _finalize0(partials, u, d_col, b):
    n, c = u.shape
    tm = 512
    return pl.pallas_call(
        _finalize0_kernel,
        out_shape=jax.ShapeDtypeStruct((n, c), jnp.float32),
        grid=(n // tm,),
        in_specs=[
            pl.BlockSpec((1, tm, c), lambda i: (0, i, 0)),
            pl.BlockSpec((1, tm, c), lambda i: (1, i, 0)),
            pl.BlockSpec((tm, c), lambda i: (i, 0)),
            pl.BlockSpec((tm, 1), lambda i: (i, 0)),
            pl.BlockSpec((1, c), lambda i: (0, 0)),
        ],
        out_specs=pl.BlockSpec((tm, c), lambda i: (i, 0)),
        compiler_params=pltpu.CompilerParams(
            dimension_semantics=("parallel",)),
    )(partials, partials, u, d_col, b)


def _finalize(partials, u, d_col, w, b, *, rescale, out_dtype):
    n, c = u.shape
    tm = 512
    return pl.pallas_call(
        functools.partial(_finalize_kernel, rescale=rescale),
        out_shape=jax.ShapeDtypeStruct((n, c), out_dtype),
        grid=(n // tm,),
        in_specs=[
            pl.BlockSpec((1, tm, c), lambda i: (0, i, 0)),
            pl.BlockSpec((1, tm, c), lambda i: (1, i, 0)),
            pl.BlockSpec((tm, c), lambda i: (i, 0)),
            pl.BlockSpec((tm, 1), lambda i: (i, 0)),
            pl.BlockSpec((c, c), lambda i: (0, 0)),
            pl.BlockSpec((1, c), lambda i: (0, 0)),
        ],
        out_specs=pl.BlockSpec((tm, c), lambda i: (i, 0)),
        compiler_params=pltpu.CompilerParams(
            dimension_semantics=("parallel",)),
    )(partials, partials, u, d_col, w, b)


def _readout(pool, h, w_lin, b_lin):
    g = pool.shape[0]
    o = w_lin.shape[1]
    vspec = pl.BlockSpec(memory_space=pltpu.MemorySpace.VMEM)
    return pl.pallas_call(
        _readout_kernel,
        out_shape=jax.ShapeDtypeStruct((g, o), jnp.float32),
        in_specs=[vspec] * 4,
        out_specs=vspec,
        compiler_params=pltpu.CompilerParams(
            vmem_limit_bytes=32 * 1024 * 1024),
    )(pool, h, w_lin, b_lin)


def kernel(x, edge_index, batch,
           w0, b0, gamma0, beta0, run_mean0, run_var0,
           w1, b1, gamma1, beta1, run_mean1, run_var1,
           w2, b2, gamma2, beta2, run_mean2, run_var2,
           lin_w, lin_b):
    n_nodes = x.shape[0]
    num_graphs = 32
    out_ch = lin_w.shape[1]

    src, dst = edge_index[0], edge_index[1]
    deg = jnp.zeros((n_nodes,), jnp.float32).at[dst].add(1.0) + 1.0
    d_inv_sqrt = 1.0 / jnp.sqrt(deg)
    d_col = d_inv_sqrt.reshape(n_nodes, 1)

    pk = (dst << (n_nodes - 1).bit_length()) | src

    ws, bs = [], []
    for (w, b, gamma, beta, mean, var) in (
            (w0, b0, gamma0, beta0, run_mean0, run_var0),
            (w1, b1, gamma1, beta1, run_mean1, run_var1),
            (w2, b2, gamma2, beta2, run_mean2, run_var2)):
        scale = gamma / jnp.sqrt(var + BN_EPS)
        ws.append((w * scale[None, :]).astype(jnp.float32))
        bs.append(((b - mean) * scale + beta).reshape(1, -1).astype(jnp.float32))

    pool = (batch[None, :] == jnp.arange(num_graphs, dtype=batch.dtype)[:, None]
            ).astype(jnp.bfloat16)

    u = _transform(x, ws[0], d_col)
    partials = _aggregate(pk, u)
    u = _finalize0(partials, u, d_col, bs[0])
    partials = _aggregate(pk, u)
    u = _finalize(partials, u, d_col, ws[1], bs[1],
                  rescale=True, out_dtype=jnp.float32)
    partials = _aggregate(pk, u)
    h3 = _finalize(partials, u, d_col, ws[2], bs[2],
                   rescale=False, out_dtype=jnp.bfloat16)
    out = _readout(pool, h3, lin_w.astype(jnp.float32),
                   lin_b.reshape(1, -1).astype(jnp.float32))
    return out[:, :out_ch]
